# baseline (speedup 1.0000x reference)
import jax
import jax.numpy as jnp
from jax.experimental import pallas as pl
from jax.experimental.pallas import tpu as pltpu

_NCH = 512
_RING = 3


def _body(x_h, pos_h, out_h, xb, ob, pb, sx, sp, so):
    batch, seq_len, d_model = x_h.shape
    n_s = seq_len // _NCH
    units = [(s, b) for s in range(n_s) for b in range(batch)]
    n_u = len(units)
    xload = [None] * n_u
    pload = [None] * n_s
    ostore = [None] * n_u

    def issue(v):
        s, b = units[v]
        if b == 0:
            pload[s] = pltpu.make_async_copy(
                pos_h.at[pl.ds(s * _NCH, _NCH)], pb.at[s], sp.at[s % 2]
            )
            pload[s].start()
        xload[v] = pltpu.make_async_copy(
            x_h.at[b, pl.ds(s * _NCH, _NCH)], xb.at[v % _RING], sx.at[v % _RING]
        )
        xload[v].start()

    for v in range(min(_RING, n_u)):
        issue(v)
    for u in range(n_u):
        s, b = units[u]
        k = u % _RING
        xload[u].wait()
        if b == 0:
            pload[s].wait()
        if u >= _RING:
            ostore[u - _RING].wait()
        ob[k, :, :] = xb[k, :, :] + pb[s, :, :]
        ostore[u] = pltpu.make_async_copy(
            ob.at[k], out_h.at[b, pl.ds(s * _NCH, _NCH)], so.at[k]
        )
        ostore[u].start()
        if u + _RING < n_u:
            issue(u + _RING)
    for u in range(n_u - _RING, n_u):
        ostore[u].wait()


def kernel(x, pos_table):
    batch, seq_len, d_model = x.shape
    n_s = seq_len // _NCH
    return pl.pallas_call(
        _body,
        in_specs=[
            pl.BlockSpec(memory_space=pl.ANY),
            pl.BlockSpec(memory_space=pl.ANY),
        ],
        out_specs=pl.BlockSpec(memory_space=pl.ANY),
        out_shape=jax.ShapeDtypeStruct(x.shape, x.dtype),
        scratch_shapes=[
            pltpu.VMEM((_RING, _NCH, d_model), jnp.float32),
            pltpu.VMEM((_RING, _NCH, d_model), jnp.float32),
            pltpu.VMEM((n_s, _NCH, d_model), jnp.float32),
            pltpu.SemaphoreType.DMA((_RING,)),
            pltpu.SemaphoreType.DMA((2,)),
            pltpu.SemaphoreType.DMA((_RING,)),
        ],
    )(x, pos_table)
